# Initial kernel scaffold; baseline (speedup 1.0000x reference)
import jax
import jax.numpy as jnp
from jax.experimental import pallas as pl


def kernel(inputs, emb_w1, emb_g1, emb_b1, emb_w2, emb_g2, emb_b2, blk0_wp, blk0_wv, blk1_wp, blk1_wv, blk2_wp, blk2_wv, fuse_w, fuse_g, fuse_b, cls_w1, cls_b1, cls_g1, cls_bb1, cls_w2, cls_b2, cls_g2, cls_bb2, cls_w3, cls_b3):
    raise NotImplementedError("write your pallas kernel here")


# speedup vs baseline: 11.8640x; 11.8640x over previous
import jax
import jax.numpy as jnp
from jax import lax
from jax.experimental import pallas as pl
from jax.experimental.pallas import tpu as pltpu
from jax.experimental.pallas import tpu_sc as plsc

F32 = jnp.float32
I32 = jnp.int32

B = 2
N = 4096
R = B * N
C = 128
K = 16
NBLK = 8
RB = R // NBLK
KNN_BLK = 256
PB = 512
NPB = R // PB


def _knn_body(ptb_ref, ptT_ref, idx_ref):
    b = pl.program_id(0)
    ptb = ptb_ref[0]
    ptT = ptT_ref[0]
    sqb = jnp.sum(ptb * ptb, axis=1)
    sqf = jnp.sum(ptT * ptT, axis=0)
    dot = jnp.dot(ptb, ptT, preferred_element_type=F32)
    d = sqb[:, None] + sqf[None, :] - 2.0 * dot
    cols = lax.broadcasted_iota(I32, (KNN_BLK, N), 1)
    off = b * N
    outs = []
    for k in range(K):
        m = jnp.min(d, axis=1)
        a = jnp.min(jnp.where(d == m[:, None], cols, N), axis=1)
        outs.append(a)
        d = jnp.where(cols == a[:, None], jnp.inf, d)
    idx_ref[0] = jnp.stack(outs, axis=1) + off


def _knn(pt_rows, ptT):
    return pl.pallas_call(
        _knn_body,
        grid=(B, N // KNN_BLK),
        in_specs=[
            pl.BlockSpec((1, KNN_BLK, 8), lambda b, r: (b, r, 0)),
            pl.BlockSpec((1, 8, N), lambda b, r: (b, 0, 0)),
        ],
        out_specs=pl.BlockSpec((1, KNN_BLK, K), lambda b, r: (b, r, 0)),
        out_shape=jax.ShapeDtypeStruct((B, N, K), I32),
    )(pt_rows, ptT)


def _e1_body(xr_ref, w1t_ref, y1_ref):
    y1_ref[...] = jnp.dot(xr_ref[...], w1t_ref[...],
                          preferred_element_type=F32)


def _e1(xr, w1t):
    return pl.pallas_call(
        _e1_body, out_shape=jax.ShapeDtypeStruct((R, C), F32),
    )(xr, w1t)


def _bn_expr(y, m, v, g, b):
    eps = 1e-5
    return (y - m) / jnp.sqrt(v + eps) * g + b


def _e2_body(y1_ref, m_ref, v_ref, g_ref, b_ref, w2t_ref, y2_ref):
    h1 = jnp.maximum(_bn_expr(y1_ref[...], m_ref[...], v_ref[...],
                              g_ref[...], b_ref[...]), 0.0)
    y2_ref[...] = jnp.dot(h1, w2t_ref[...], preferred_element_type=F32)


def _e2(y1, m, v, g, b, w2t):
    return pl.pallas_call(
        _e2_body, out_shape=jax.ShapeDtypeStruct((R, C), F32),
    )(y1, m, v, g, b, w2t)


def _e3_body(y2_ref, m_ref, v_ref, g_ref, b_ref, x0_ref):
    x0_ref[...] = jnp.maximum(
        _bn_expr(y2_ref[...], m_ref[...], v_ref[...], g_ref[...],
                 b_ref[...]), 0.0)


def _e3(y2, m, v, g, b):
    return pl.pallas_call(
        _e3_body, out_shape=jax.ShapeDtypeStruct((R, C), F32),
    )(y2, m, v, g, b)


def _stats(rows, c):
    t = jnp.transpose(rows.reshape(B, N, c), (0, 2, 1))
    m = jnp.mean(t, axis=(0, 2))
    v = jnp.var(t, axis=(0, 2))
    return m[None, :], v[None, :]


_NW = 32
_CP = 8
_PW = R // _NW
_NCHUNK = _PW // _CP


def _scgather_body(x_hbm, idx_hbm, g_hbm, idx_v, rows_v, sem):
    wid = lax.axis_index("s") * 2 + lax.axis_index("c")
    base = wid * _PW

    def chunk(ci, carry):
        cbase = base + ci * _CP
        pltpu.sync_copy(idx_hbm.at[pl.ds(cbase * K, _CP * K)], idx_v)
        pltpu.async_copy(x_hbm.at[idx_v], rows_v, sem).wait()
        pltpu.sync_copy(rows_v, g_hbm.at[pl.ds(cbase * K, _CP * K)])
        return carry

    lax.fori_loop(0, _NCHUNK, chunk, 0, unroll=False)


def _scgather(x, idxf):
    mesh = plsc.VectorSubcoreMesh(core_axis_name="c", subcore_axis_name="s",
                                  num_cores=2, num_subcores=16)
    fn = pl.kernel(
        _scgather_body,
        out_type=jax.ShapeDtypeStruct((R * K, C), F32),
        mesh=mesh,
        scratch_types=[
            pltpu.VMEM((_CP * K,), I32),
            pltpu.VMEM((_CP * K, C), F32),
            pltpu.SemaphoreType.DMA,
        ],
    )
    return fn(x, idxf)


def _blk_body(x_ref, g_ref, wpt_ref, vt_ref, out_ref):
    xb = x_ref[...]
    g3 = g_ref[...].reshape(PB, K, C)
    wpt = wpt_ref[...]
    point = None
    for j in range(K):
        edge = g3[:, j, :] - xb
        cat = jnp.concatenate([xb, edge], axis=1)
        h = jnp.maximum(
            jnp.dot(cat, wpt, preferred_element_type=F32), 0.0)
        point = h if point is None else jnp.maximum(point, h)
    vox = jnp.maximum(
        jnp.dot(xb, vt_ref[...], preferred_element_type=F32), 0.0)
    out_ref[...] = point + vox + xb


def _blk(x, g, wpt, vt):
    return pl.pallas_call(
        _blk_body,
        grid=(NPB,),
        in_specs=[
            pl.BlockSpec((PB, C), lambda r: (r, 0)),
            pl.BlockSpec((PB * K, C), lambda r: (r, 0)),
            pl.BlockSpec((2 * C, C), lambda r: (0, 0)),
            pl.BlockSpec((C, C), lambda r: (0, 0)),
        ],
        out_specs=pl.BlockSpec((PB, C), lambda r: (r, 0)),
        out_shape=jax.ShapeDtypeStruct((R, C), F32),
    )(x, g, wpt, vt)


def _f1_body(x1_ref, x2_ref, x3_ref, f1t_ref, f2t_ref, f3t_ref, y_ref):
    y_ref[...] = (
        jnp.dot(x1_ref[...], f1t_ref[...], preferred_element_type=F32)
        + jnp.dot(x2_ref[...], f2t_ref[...], preferred_element_type=F32)
        + jnp.dot(x3_ref[...], f3t_ref[...], preferred_element_type=F32))


def _f1(x1, x2, x3, f1t, f2t, f3t):
    return pl.pallas_call(
        _f1_body,
        grid=(NBLK,),
        in_specs=[pl.BlockSpec((RB, C), lambda r: (r, 0))] * 3 +
                 [pl.BlockSpec((C, 1024), lambda r: (0, 0))] * 3,
        out_specs=pl.BlockSpec((RB, 1024), lambda r: (r, 0)),
        out_shape=jax.ShapeDtypeStruct((R, 1024), F32),
    )(x1, x2, x3, f1t, f2t, f3t)


def _f2_body(y_ref, m_ref, v_ref, g_ref, b_ref, z_ref, zmax_ref, zsum_ref):
    zz = _bn_expr(y_ref[0], m_ref[...], v_ref[...], g_ref[...], b_ref[...])
    z = jnp.where(zz > 0, zz, 0.2 * zz)
    z_ref[0] = z

    @pl.when(pl.program_id(1) == 0)
    def _init():
        zmax_ref[...] = jnp.full_like(zmax_ref, -jnp.inf)
        zsum_ref[...] = jnp.zeros_like(zsum_ref)

    zmax_ref[0] = jnp.maximum(zmax_ref[0],
                              jnp.max(z, axis=0, keepdims=True))
    zsum_ref[0] += jnp.sum(z, axis=0, keepdims=True)


def _f2(y, m, v, g, b):
    nbb = N // RB
    return pl.pallas_call(
        _f2_body,
        grid=(B, nbb),
        in_specs=[
            pl.BlockSpec((1, RB, 1024), lambda b_, r: (b_, r, 0)),
            pl.BlockSpec((1, 1024), lambda b_, r: (0, 0)),
            pl.BlockSpec((1, 1024), lambda b_, r: (0, 0)),
            pl.BlockSpec((1, 1024), lambda b_, r: (0, 0)),
            pl.BlockSpec((1, 1024), lambda b_, r: (0, 0)),
        ],
        out_specs=[
            pl.BlockSpec((1, RB, 1024), lambda b_, r: (b_, r, 0)),
            pl.BlockSpec((1, 1, 1024), lambda b_, r: (b_, 0, 0)),
            pl.BlockSpec((1, 1, 1024), lambda b_, r: (b_, 0, 0)),
        ],
        out_shape=[
            jax.ShapeDtypeStruct((B, N, 1024), F32),
            jax.ShapeDtypeStruct((B, 1, 1024), F32),
            jax.ShapeDtypeStruct((B, 1, 1024), F32),
        ],
    )(y.reshape(B, N, 1024), m, v, g, b)


def _f3_body(z_ref, zmax_ref, zsum_ref, wat_ref, wbt_ref, wct_ref, cb_ref,
             s1_ref):
    zmax = zmax_ref[0, 0][None, :]
    zavg = zsum_ref[0, 0][None, :] / N
    bias = (jnp.dot(zmax, wbt_ref[...], preferred_element_type=F32)
            + jnp.dot(zavg, wct_ref[...], preferred_element_type=F32)
            + cb_ref[...])
    s1_ref[0] = jnp.dot(z_ref[0], wat_ref[...],
                        preferred_element_type=F32) + bias


def _f3(z, zmax, zsum, wat, wbt, wct, cb):
    nbb = N // RB
    return pl.pallas_call(
        _f3_body,
        grid=(B, nbb),
        in_specs=[
            pl.BlockSpec((1, RB, 1024), lambda b_, r: (b_, r, 0)),
            pl.BlockSpec((1, 1, 1024), lambda b_, r: (b_, 0, 0)),
            pl.BlockSpec((1, 1, 1024), lambda b_, r: (b_, 0, 0)),
            pl.BlockSpec((1024, 512), lambda b_, r: (0, 0)),
            pl.BlockSpec((1024, 512), lambda b_, r: (0, 0)),
            pl.BlockSpec((1024, 512), lambda b_, r: (0, 0)),
            pl.BlockSpec((1, 512), lambda b_, r: (0, 0)),
        ],
        out_specs=pl.BlockSpec((1, RB, 512), lambda b_, r: (b_, r, 0)),
        out_shape=jax.ShapeDtypeStruct((B, N, 512), F32),
    )(z, zmax, zsum, wat, wbt, wct, cb)


def _f4_body(s1_ref, m_ref, v_ref, g_ref, bb_ref, w2t_ref, cb2_ref, s2_ref):
    u1 = jnp.maximum(_bn_expr(s1_ref[...], m_ref[...], v_ref[...],
                              g_ref[...], bb_ref[...]), 0.0)
    s2_ref[...] = jnp.dot(u1, w2t_ref[...],
                          preferred_element_type=F32) + cb2_ref[...]


def _f4(s1, m, v, g, bb, w2t, cb2):
    return pl.pallas_call(
        _f4_body,
        grid=(NBLK,),
        in_specs=[
            pl.BlockSpec((RB, 512), lambda r: (r, 0)),
            pl.BlockSpec((1, 512), lambda r: (0, 0)),
            pl.BlockSpec((1, 512), lambda r: (0, 0)),
            pl.BlockSpec((1, 512), lambda r: (0, 0)),
            pl.BlockSpec((1, 512), lambda r: (0, 0)),
            pl.BlockSpec((512, 256), lambda r: (0, 0)),
            pl.BlockSpec((1, 256), lambda r: (0, 0)),
        ],
        out_specs=pl.BlockSpec((RB, 256), lambda r: (r, 0)),
        out_shape=jax.ShapeDtypeStruct((R, 256), F32),
    )(s1, m, v, g, bb, w2t, cb2)


def _f5_body(s2_ref, m_ref, v_ref, g_ref, bb_ref, w3t_ref, cb3_ref, o_ref):
    u2 = jnp.maximum(_bn_expr(s2_ref[...], m_ref[...], v_ref[...],
                              g_ref[...], bb_ref[...]), 0.0)
    o_ref[...] = jnp.dot(u2, w3t_ref[...],
                         preferred_element_type=F32) + cb3_ref[...]


def _f5(s2, m, v, g, bb, w3t, cb3):
    return pl.pallas_call(
        _f5_body,
        grid=(NBLK,),
        in_specs=[
            pl.BlockSpec((RB, 256), lambda r: (r, 0)),
            pl.BlockSpec((1, 256), lambda r: (0, 0)),
            pl.BlockSpec((1, 256), lambda r: (0, 0)),
            pl.BlockSpec((1, 256), lambda r: (0, 0)),
            pl.BlockSpec((1, 256), lambda r: (0, 0)),
            pl.BlockSpec((256, 128), lambda r: (0, 0)),
            pl.BlockSpec((1, 128), lambda r: (0, 0)),
        ],
        out_specs=pl.BlockSpec((RB, 128), lambda r: (r, 0)),
        out_shape=jax.ShapeDtypeStruct((R, 128), F32),
    )(s2, m, v, g, bb, w3t, cb3)


def kernel(inputs, emb_w1, emb_g1, emb_b1, emb_w2, emb_g2, emb_b2,
           blk0_wp, blk0_wv, blk1_wp, blk1_wv, blk2_wp, blk2_wv,
           fuse_w, fuse_g, fuse_b,
           cls_w1, cls_b1, cls_g1, cls_bb1,
           cls_w2, cls_b2, cls_g2, cls_bb2,
           cls_w3, cls_b3):
    x9 = inputs[:, :9, :]
    xyzT = jnp.transpose(x9[:, :3, :], (0, 2, 1))
    pt_rows = jnp.pad(xyzT, ((0, 0), (0, 0), (0, 5)))
    ptT = jnp.pad(x9[:, :3, :], ((0, 0), (0, 5), (0, 0)))

    idx = _knn(pt_rows, ptT)
    idxf = idx.reshape(R * K)

    xr = jnp.pad(jnp.transpose(x9, (0, 2, 1)).reshape(R, 9),
                 ((0, 0), (0, 7)))
    w1t = jnp.pad(emb_w1.T, ((0, 7), (0, 0)))
    y1 = _e1(xr, w1t)
    m1, v1 = _stats(y1, C)
    y2 = _e2(y1, m1, v1, emb_g1[None, :], emb_b1[None, :], emb_w2.T)
    m2, v2 = _stats(y2, C)
    x0 = _e3(y2, m2, v2, emb_g2[None, :], emb_b2[None, :])

    wps = (blk0_wp, blk1_wp, blk2_wp)
    wvs = (blk0_wv, blk1_wv, blk2_wv)
    xcur = x0
    xs = []
    for i in range(3):
        g = _scgather(xcur, idxf)
        xcur = _blk(xcur, g, wps[i].T, wvs[i].T)
        xs.append(xcur)
    x1, x2, x3 = xs

    f1t = fuse_w[:, :C].T
    f2t = fuse_w[:, C:2 * C].T
    f3t = fuse_w[:, 2 * C:].T
    y = _f1(x1, x2, x3, f1t, f2t, f3t)
    my, vy = _stats(y, 1024)

    z, zmax, zsum = _f2(y, my, vy, fuse_g[None, :], fuse_b[None, :])

    wat = cls_w1[:, :1024].T
    wbt = cls_w1[:, 1024:2048].T
    wct = cls_w1[:, 2048:].T
    s1 = _f3(z, zmax, zsum, wat, wbt, wct, cls_b1[None, :])
    ms1, vs1 = _stats(s1.reshape(R, 512), 512)

    s2 = _f4(s1.reshape(R, 512), ms1, vs1, cls_g1[None, :], cls_bb1[None, :],
             cls_w2.T, cls_b2[None, :])
    ms2, vs2 = _stats(s2, 256)

    w3t = jnp.pad(cls_w3.T, ((0, 0), (0, 128 - 13)))
    cb3 = jnp.pad(cls_b3, ((0, 128 - 13)))[None, :]
    o = _f5(s2, ms2, vs2, cls_g2[None, :], cls_bb2[None, :], w3t, cb3)

    return jnp.transpose(o[:, :13].reshape(B, N, 13), (0, 2, 1))
